# baseline (speedup 1.0000x reference)
import functools

import jax
import jax.numpy as jnp
from jax import lax
from jax.experimental import pallas as pl
from jax.experimental.pallas import tpu as pltpu
from jax.experimental.pallas import tpu_sc as plsc

_NUM_CORES = 2
_NUM_SUBCORES = 16
_NUM_WORKERS = _NUM_CORES * _NUM_SUBCORES
_LANES = 16


@functools.lru_cache(maxsize=None)
def _make_broadcast_kernel(E: int, D: int):
    rows_per_worker = E // _NUM_WORKERS
    assert E % _NUM_WORKERS == 0 and D % _LANES == 0

    chunk_rows = 400
    while rows_per_worker % chunk_rows or chunk_rows % 8:
        chunk_rows //= 2
    n_chunks = rows_per_worker // chunk_rows

    mesh = plsc.VectorSubcoreMesh(core_axis_name="c", subcore_axis_name="s")

    @functools.partial(
        pl.kernel,
        mesh=mesh,
        out_type=jax.ShapeDtypeStruct((E, D), jnp.float32),
        scratch_types=[
            pltpu.VMEM((chunk_rows, D), jnp.float32),
            pltpu.SemaphoreType.DMA,
        ],
    )
    def bcast(table_hbm, out_hbm, buf, sem):
        wid = lax.axis_index("s") * _NUM_CORES + lax.axis_index("c")
        base = wid * rows_per_worker

        pltpu.sync_copy(table_hbm, buf.at[pl.ds(0, 1)])

        regs = [buf[0, pl.ds(j * _LANES, _LANES)] for j in range(D // _LANES)]

        def fill(i, carry):
            for j in range(D // _LANES):
                buf[i, pl.ds(j * _LANES, _LANES)] = regs[j]
            return carry

        lax.fori_loop(1, chunk_rows, fill, 0, unroll=4)

        copies = [
            pltpu.async_copy(
                buf, out_hbm.at[pl.ds(base + i * chunk_rows, chunk_rows)], sem
            )
            for i in range(n_chunks)
        ]
        for c in copies:
            c.wait()

    return bcast


def kernel(edge_index, table):
    E = edge_index.shape[1]
    D = table.shape[1]
    return _make_broadcast_kernel(E, D)(table.astype(jnp.float32))


# speedup vs baseline: 1.0107x; 1.0107x over previous
import functools

import jax
import jax.numpy as jnp
from jax import lax
from jax.experimental import pallas as pl
from jax.experimental.pallas import tpu as pltpu
from jax.experimental.pallas import tpu_sc as plsc

_NUM_CORES = 2
_NUM_SUBCORES = 16
_NUM_WORKERS = _NUM_CORES * _NUM_SUBCORES
_LANES = 16


@functools.lru_cache(maxsize=None)
def _make_broadcast_kernel(E: int, D: int):
    rows_per_worker = E // _NUM_WORKERS
    assert E % _NUM_WORKERS == 0 and D % _LANES == 0

    chunk_rows = 200
    while rows_per_worker % chunk_rows or chunk_rows % 8:
        chunk_rows //= 2
    n_chunks = rows_per_worker // chunk_rows

    mesh = plsc.VectorSubcoreMesh(core_axis_name="c", subcore_axis_name="s")

    @functools.partial(
        pl.kernel,
        mesh=mesh,
        out_type=jax.ShapeDtypeStruct((E, D), jnp.float32),
        scratch_types=[
            pltpu.VMEM((chunk_rows, D), jnp.float32),
            pltpu.SemaphoreType.DMA,
        ],
    )
    def bcast(table_hbm, out_hbm, buf, sem):
        wid = lax.axis_index("s") * _NUM_CORES + lax.axis_index("c")
        base = wid * rows_per_worker

        pltpu.sync_copy(table_hbm, buf.at[pl.ds(0, 1)])

        regs = [buf[0, pl.ds(j * _LANES, _LANES)] for j in range(D // _LANES)]

        def fill(i, carry):
            for j in range(D // _LANES):
                buf[i, pl.ds(j * _LANES, _LANES)] = regs[j]
            return carry

        lax.fori_loop(1, chunk_rows, fill, 0, unroll=8)

        copies = [
            pltpu.async_copy(
                buf, out_hbm.at[pl.ds(base + i * chunk_rows, chunk_rows)], sem
            )
            for i in range(n_chunks)
        ]
        for c in copies:
            c.wait()

    return bcast


def kernel(edge_index, table):
    E = edge_index.shape[1]
    D = table.shape[1]
    return _make_broadcast_kernel(E, D)(table)
